# baseline (speedup 1.0000x reference)
import jax
import jax.numpy as jnp
from jax import lax
from jax.experimental import pallas as pl

_ROWS2D = 32


def _predict_kernel(fb_ref, ent2d_ref, is02d_ref, ent_memo_ref,
                    embed_memo_ref, memo_pred_ref):
    f32 = jnp.float32
    bf16 = jnp.bfloat16
    i32 = jnp.int32
    fb = fb_ref[...]
    Bn = fb.shape[0]
    Mn = ent_memo_ref.shape[1]
    R2, L2 = _ROWS2D, Bn // _ROWS2D

    ent2d = ent2d_ref[...]
    is02d = is02d_ref[...] != 0

    inf = jnp.array(jnp.inf, f32)
    idx2d = (lax.broadcasted_iota(i32, (R2, L2), 0) * L2
             + lax.broadcasted_iota(i32, (R2, L2), 1))

    tri_l = (lax.broadcasted_iota(i32, (L2, L2), 0)
             < lax.broadcasted_iota(i32, (L2, L2), 1)).astype(f32)
    tri_r = (lax.broadcasted_iota(i32, (R2, R2), 0)
             > lax.broadcasted_iota(i32, (R2, R2), 1)).astype(f32)
    jj_r = lax.broadcasted_iota(i32, (Mn, Mn), 0)
    jj_c = lax.broadcasted_iota(i32, (Mn, Mn), 1)
    tri_m = (jj_c < jj_r).astype(f32)

    sels = []
    for c in range(2):
        m_2d = is02d if c == 0 else jnp.logical_not(is02d)
        sel2d = jnp.where(m_2d, ent2d, inf)
        bits2d = lax.bitcast_convert_type(sel2d + f32(0.0), i32)

        v_row = ent_memo_ref[c, :][None, :]
        v_col = ent_memo_ref[c, :][:, None]
        sel_flat = sel2d.reshape(1, Bn)

        cnt_less = jnp.sum((sel_flat < v_col).astype(f32), axis=1,
                           keepdims=True)
        gtT = (v_row > v_col) | ((v_row == v_col) & (jj_c < jj_r))
        rank = jnp.sum(gtT.astype(f32), axis=1, keepdims=True)
        replaced = cnt_less > rank
        repl_f = replaced.astype(f32)
        K = jnp.sum(repl_f).astype(i32)

        rpos = lax.dot_general(tri_m, repl_f, (((1,), (0,)), ((), ())),
                               preferred_element_type=f32).astype(i32)

        def val_body(_, lh):
            lo, hi = lh
            mid = (lo + hi) // 2
            cnt = jnp.sum((bits2d <= mid).astype(i32))
            ge = cnt >= K
            return (jnp.where(ge, lo, mid + 1), jnp.where(ge, mid, hi))

        lo0 = jnp.array(0, i32)
        _, vk = lax.fori_loop(0, 31, val_body, (lo0, jnp.array(0x7f800000, i32)))
        n_less = jnp.sum((bits2d < vk).astype(i32))
        need = K - n_less
        tie = bits2d == vk

        def idx_body(_, lh):
            lo, hi = lh
            mid = (lo + hi) // 2
            cnt = jnp.sum((tie & (idx2d <= mid)).astype(i32))
            ge = cnt >= need
            return (jnp.where(ge, lo, mid + 1), jnp.where(ge, mid, hi))

        _, rk = lax.fori_loop(0, 12, idx_body, (lo0, jnp.array(Bn - 1, i32)))
        w2d = ((bits2d < vk) | (tie & (idx2d <= rk))) & (K > 0)
        w2d_f = w2d.astype(f32)

        p_lane = lax.dot_general(w2d_f, tri_l, (((1,), (0,)), ((), ())),
                                 preferred_element_type=f32)
        s_row = jnp.sum(w2d_f, axis=1, keepdims=True)
        p_row = lax.dot_general(tri_r, s_row, (((1,), (0,)), ((), ())),
                                preferred_element_type=f32)
        pos2d = (p_lane + p_row).astype(i32)

        pos_flat = jnp.where(w2d, pos2d, -1).reshape(1, Bn)
        rpos_m = jnp.where(replaced, rpos, -2)
        sels.append((replaced, rpos_m, pos_flat))

    cosins = []
    for c in range(2):
        replaced, rpos_m, pos_flat = sels[c]
        scat = (rpos_m == pos_flat).astype(f32).astype(bf16)
        wrows = lax.dot_general(scat, fb, (((1,), (0,)), ((), ())),
                                preferred_element_type=f32)
        newmemo = jnp.where(replaced, wrows, embed_memo_ref[c])
        nm_b = newmemo.astype(bf16)

        p_sim = lax.dot_general(fb, nm_b, (((1,), (1,)), ((), ())),
                                preferred_element_type=f32)
        cosins.append(jnp.sum(p_sim, axis=1, keepdims=True))

    c0, c1 = cosins
    cm = jnp.maximum(c0, c1)
    q0 = jnp.exp(c0 - cm)
    q1 = jnp.exp(c1 - cm)
    qz = q0 + q1
    memo_pred_ref[:, 0:1] = q0 / qz
    memo_pred_ref[:, 1:2] = q1 / qz


@jax.jit
def kernel(fused_embeds, logits, entropy_memo, embed_memo):
    b = fused_embeds.shape[0]
    pred = jax.nn.softmax(logits, axis=-1)
    log_pred = jax.nn.log_softmax(logits, axis=-1)
    entropy = -jnp.sum(pred * log_pred, axis=-1)
    pseudo_y = jnp.argmax(pred, axis=-1)
    is0 = (pseudo_y == 0).astype(jnp.float32)
    memo_pred = pl.pallas_call(
        _predict_kernel,
        out_shape=jax.ShapeDtypeStruct((b, 2), jnp.float32),
    )(fused_embeds.astype(jnp.bfloat16),
      entropy.reshape(_ROWS2D, b // _ROWS2D),
      is0.reshape(_ROWS2D, b // _ROWS2D),
      entropy_memo, embed_memo)
    return memo_pred, pred, entropy


# speedup vs baseline: 1.1163x; 1.1163x over previous
import jax
import jax.numpy as jnp
from jax import lax
from jax.experimental import pallas as pl

_ROWS2D = 32


def _predict_kernel(fused_ref, ent2d_ref, is02d_ref, ent_memo_ref,
                    embed_memo_ref, memo_pred_ref):
    f32 = jnp.float32
    bf16 = jnp.bfloat16
    i32 = jnp.int32
    fused = fused_ref[...]
    Bn = fused.shape[0]
    Mn = ent_memo_ref.shape[1]
    R2, L2 = _ROWS2D, Bn // _ROWS2D

    fb = fused.astype(bf16)
    ent2d = ent2d_ref[...]
    is02d = is02d_ref[...] != 0

    inf = jnp.array(jnp.inf, f32)
    idx2d = (lax.broadcasted_iota(i32, (R2, L2), 0) * L2
             + lax.broadcasted_iota(i32, (R2, L2), 1))

    tri_l = (lax.broadcasted_iota(i32, (L2, L2), 0)
             < lax.broadcasted_iota(i32, (L2, L2), 1)).astype(f32)
    tri_r = (lax.broadcasted_iota(i32, (R2, R2), 0)
             > lax.broadcasted_iota(i32, (R2, R2), 1)).astype(f32)
    jj_r = lax.broadcasted_iota(i32, (Mn, Mn), 0)
    jj_c = lax.broadcasted_iota(i32, (Mn, Mn), 1)
    tri_m = (jj_c < jj_r).astype(f32)

    sels = []
    for c in range(2):
        m_2d = is02d if c == 0 else jnp.logical_not(is02d)
        sel2d = jnp.where(m_2d, ent2d, inf)
        bits2d = lax.bitcast_convert_type(sel2d + f32(0.0), i32)

        v_row = ent_memo_ref[c, :][None, :]
        v_col = ent_memo_ref[c, :][:, None]
        sel_flat = sel2d.reshape(1, Bn)

        cnt_less = jnp.sum((sel_flat < v_col).astype(f32), axis=1,
                           keepdims=True)
        gtT = (v_row > v_col) | ((v_row == v_col) & (jj_c < jj_r))
        rank = jnp.sum(gtT.astype(f32), axis=1, keepdims=True)
        replaced = cnt_less > rank
        repl_f = replaced.astype(f32)
        K = jnp.sum(repl_f).astype(i32)

        rpos = lax.dot_general(tri_m, repl_f, (((1,), (0,)), ((), ())),
                               preferred_element_type=f32).astype(i32)

        def val_body(_, lh):
            lo, hi = lh
            mid = (lo + hi) // 2
            cnt = jnp.sum((bits2d <= mid).astype(i32))
            ge = cnt >= K
            return (jnp.where(ge, lo, mid + 1), jnp.where(ge, mid, hi))

        lo0 = jnp.array(0, i32)
        _, vk = lax.fori_loop(0, 31, val_body, (lo0, jnp.array(0x7f800000, i32)))
        n_less = jnp.sum((bits2d < vk).astype(i32))
        need = K - n_less
        tie = bits2d == vk

        def idx_body(_, lh):
            lo, hi = lh
            mid = (lo + hi) // 2
            cnt = jnp.sum((tie & (idx2d <= mid)).astype(i32))
            ge = cnt >= need
            return (jnp.where(ge, lo, mid + 1), jnp.where(ge, mid, hi))

        _, rk = lax.fori_loop(0, 12, idx_body, (lo0, jnp.array(Bn - 1, i32)))
        w2d = ((bits2d < vk) | (tie & (idx2d <= rk))) & (K > 0)
        w2d_f = w2d.astype(f32)

        p_lane = lax.dot_general(w2d_f, tri_l, (((1,), (0,)), ((), ())),
                                 preferred_element_type=f32)
        s_row = jnp.sum(w2d_f, axis=1, keepdims=True)
        p_row = lax.dot_general(tri_r, s_row, (((1,), (0,)), ((), ())),
                                preferred_element_type=f32)
        pos2d = (p_lane + p_row).astype(i32)

        pos_flat = jnp.where(w2d, pos2d, -1).reshape(1, Bn)
        rpos_m = jnp.where(replaced, rpos, -2)
        sels.append((replaced, rpos_m, pos_flat))

    cosins = []
    for c in range(2):
        replaced, rpos_m, pos_flat = sels[c]
        scat = (rpos_m == pos_flat).astype(f32).astype(bf16)
        wrows = lax.dot_general(scat, fb, (((1,), (0,)), ((), ())),
                                preferred_element_type=f32)
        newmemo = jnp.where(replaced, wrows, embed_memo_ref[c])
        nm_b = newmemo.astype(bf16)

        p_sim = lax.dot_general(fb, nm_b, (((1,), (1,)), ((), ())),
                                preferred_element_type=f32)
        cosins.append(jnp.sum(p_sim, axis=1, keepdims=True))

    c0, c1 = cosins
    cm = jnp.maximum(c0, c1)
    q0 = jnp.exp(c0 - cm)
    q1 = jnp.exp(c1 - cm)
    qz = q0 + q1
    memo_pred_ref[:, 0:1] = q0 / qz
    memo_pred_ref[:, 1:2] = q1 / qz


@jax.jit
def kernel(fused_embeds, logits, entropy_memo, embed_memo):
    b = fused_embeds.shape[0]
    pred = jax.nn.softmax(logits, axis=-1)
    log_pred = jax.nn.log_softmax(logits, axis=-1)
    entropy = -jnp.sum(pred * log_pred, axis=-1)
    pseudo_y = jnp.argmax(pred, axis=-1)
    is0 = (pseudo_y == 0).astype(jnp.float32)
    memo_pred = pl.pallas_call(
        _predict_kernel,
        out_shape=jax.ShapeDtypeStruct((b, 2), jnp.float32),
    )(fused_embeds,
      entropy.reshape(_ROWS2D, b // _ROWS2D),
      is0.reshape(_ROWS2D, b // _ROWS2D),
      entropy_memo, embed_memo)
    return memo_pred, pred, entropy
